# baseline (speedup 1.0000x reference)
import jax
import jax.numpy as jnp
from jax.experimental import pallas as pl
from jax.experimental.pallas import tpu as pltpu

_N = 512
_R_MAX = 5.0
_BESSEL = 8
_F_S = 32
_F_V = 8
_L = 2

_KPHI = jnp.pi / (2.0 * _R_MAX)
_HP2 = (0.5 * jnp.pi) ** 2
_SIN_C = (-1.0 / 39916800.0, 1.0 / 362880.0, -1.0 / 5040.0,
          1.0 / 120.0, -1.0 / 6.0, 1.0)
_COS_C = (1.0 / 479001600.0, -1.0 / 3628800.0, 1.0 / 40320.0,
          -1.0 / 720.0, 1.0 / 24.0, -0.5, 1.0)


def _dot(a, b, prec=jax.lax.Precision.DEFAULT):
    return jax.lax.dot_general(
        a, b, (((1,), (0,)), ((), ())),
        precision=prec, preferred_element_type=jnp.float32)


def _horner(coeffs, u):
    acc = jnp.full_like(u, coeffs[0])
    for c in coeffs[1:]:
        acc = acc * u + c
    return acc


def _mace_body(x_ref, xT_ref, we_ref, wrs_ref, wrv_ref, wvs_ref, ws_ref,
               wv_ref, wrds_ref, wrdv_ref, out_ref):
    n = _N
    f32 = jnp.float32
    x = x_ref[:]
    xT = xT_ref[:]

    gram = _dot(x, xT, jax.lax.Precision.HIGHEST)
    sq_i = jnp.sum(x * x, axis=1, keepdims=True)
    sq_j = jnp.sum(xT * xT, axis=0, keepdims=True)
    r2 = jnp.maximum(sq_i + sq_j - (gram + gram), 1e-4)
    inv_r = jax.lax.rsqrt(r2)

    u_raw = (_KPHI * _KPHI) * r2
    u = jnp.minimum(u_raw, _HP2)
    psin = _horner(_SIN_C, u)
    cp = _horner(_COS_C, u)
    env = cp * cp
    c1 = env + env - 1.0
    t1 = (2.0 * _KPHI) * psin * cp

    rows = jax.lax.broadcasted_iota(jnp.int32, (n, n), 0)
    cols = jax.lax.broadcasted_iota(jnp.int32, (n, n), 1)
    offdiag = rows != cols
    pref = jnp.where(
        offdiag & (u_raw < _HP2), jnp.sqrt(2.0 / _R_MAX) * env, 0.0)
    inv_r_m = jnp.where(offdiag, inv_r, 0.0)

    two_c1 = c1 + c1
    rbs = []
    t_prev = jnp.zeros_like(t1)
    t_cur = t1
    for _ in range(_BESSEL):
        rbs.append((pref * t_cur).astype(jnp.bfloat16))
        t_prev, t_cur = t_cur, two_c1 * t_cur - t_prev

    h_s = we_ref[:] + jnp.zeros((n, _F_S), f32)
    hv = jnp.zeros((n, 3 * _F_V), f32)
    inv_n = 1.0 / float(n)
    xc = [x[:, c:c + 1] for c in range(3)]
    xrep = jnp.concatenate(
        [jnp.broadcast_to(xc[c], (n, _F_V)) for c in range(3)], axis=1)
    zero88 = jnp.zeros((_F_V, _F_V), f32)

    for l in range(_L):
        w_rs = wrs_ref[l]
        w_rv = wrv_ref[l]
        S = jnp.concatenate([w_rs, w_rv, w_rv, w_rv], axis=1)
        M = jnp.concatenate([h_s, hv], axis=1)
        agg = _dot(rbs[0], (M * S[0:1, :]).astype(jnp.bfloat16))
        for b in range(1, _BESSEL):
            agg = agg + _dot(rbs[b], (M * S[b:b + 1, :]).astype(jnp.bfloat16))
        g = _dot(h_s, wvs_ref[l])
        P = jnp.concatenate([g * xc[0], g * xc[1], g * xc[2], g], axis=1)
        Q = _dot(inv_r_m, P)
        qg = jnp.concatenate([Q[:, 3 * _F_V:]] * 3, axis=1)
        av = (agg[:, _F_S:] + xrep * qg - Q[:, :3 * _F_V]) * inv_n
        h_s = h_s + _dot(agg[:, :_F_S] * inv_n, ws_ref[l])
        w_v = wv_ref[l]
        w_v_bd = jnp.concatenate([
            jnp.concatenate([w_v, zero88, zero88], axis=1),
            jnp.concatenate([zero88, w_v, zero88], axis=1),
            jnp.concatenate([zero88, zero88, w_v], axis=1)], axis=0)
        hv = hv + _dot(av, w_v_bd)

    sout = _dot(h_s, wrds_ref[:])
    wrdv = wrdv_ref[:]
    w8 = jax.lax.broadcasted_iota(jnp.int32, (_F_V, 3 * _F_V), 0)
    j24 = jax.lax.broadcasted_iota(jnp.int32, (_F_V, 3 * _F_V), 1)
    big_w = jnp.concatenate(
        [_dot(wrdv, jnp.where(j24 == 3 * w8 + c, 1.0, 0.0),
              jax.lax.Precision.HIGHEST) for c in range(3)], axis=0)
    com = jnp.sum(x, axis=0, keepdims=True) * inv_n
    com24 = jnp.concatenate([com] * _F_V, axis=1)
    out_ref[:] = jnp.concatenate([sout, _dot(hv, big_w) + com24], axis=1)


_call = pl.pallas_call(
    _mace_body,
    out_shape=jax.ShapeDtypeStruct((_N, _F_S + 3 * _F_V), jnp.float32),
    compiler_params=pltpu.CompilerParams(vmem_limit_bytes=100 * 1024 * 1024),
)


def kernel(x, W_embed, W_rad_s, W_rad_v, W_vs, W_s, W_v, W_read_s, W_read_v):
    xT = x.T
    out = _call(
        x, xT, W_embed, W_rad_s, W_rad_v, W_vs, W_s, W_v, W_read_s, W_read_v)
    return (out[:, _F_S:].reshape(_N, _F_V, 3), out[:, :_F_S])


# speedup vs baseline: 1.0284x; 1.0284x over previous
import jax
import jax.numpy as jnp
from jax.experimental import pallas as pl
from jax.experimental.pallas import tpu as pltpu

_N = 512
_R_MAX = 5.0
_BESSEL = 8
_F_S = 32
_F_V = 8
_L = 2

_KPHI = jnp.pi / (2.0 * _R_MAX)
_HP2 = (0.5 * jnp.pi) ** 2
_SIN_C = (-1.0 / 39916800.0, 1.0 / 362880.0, -1.0 / 5040.0,
          1.0 / 120.0, -1.0 / 6.0, 1.0)
_COS_C = (1.0 / 479001600.0, -1.0 / 3628800.0, 1.0 / 40320.0,
          -1.0 / 720.0, 1.0 / 24.0, -0.5, 1.0)


def _dot(a, b, prec=jax.lax.Precision.DEFAULT):
    return jax.lax.dot_general(
        a, b, (((1,), (0,)), ((), ())),
        precision=prec, preferred_element_type=jnp.float32)


def _horner(coeffs, u):
    acc = jnp.full_like(u, coeffs[0])
    for c in coeffs[1:]:
        acc = acc * u + c
    return acc


def _mace_body(x_ref, xT_ref, we_ref, wrs_ref, wrv_ref, wvs_ref, ws_ref,
               wv_ref, wrds_ref, wrdv_ref, vout_ref, sout_ref):
    n = _N
    f32 = jnp.float32
    x = x_ref[:]
    xT = xT_ref[:]

    gram = _dot(x, xT, jax.lax.Precision.HIGHEST)
    sq_i = jnp.sum(x * x, axis=1, keepdims=True)
    sq_j = jnp.sum(xT * xT, axis=0, keepdims=True)
    r2 = jnp.maximum(sq_i + sq_j - (gram + gram), 1e-4)
    inv_r = jax.lax.rsqrt(r2)

    u_raw = (_KPHI * _KPHI) * r2
    u = jnp.minimum(u_raw, _HP2)
    psin = _horner(_SIN_C, u)
    cp = _horner(_COS_C, u)
    env = cp * cp
    c1 = env + env - 1.0
    t1 = (2.0 * _KPHI) * psin * cp

    rows = jax.lax.broadcasted_iota(jnp.int32, (n, n), 0)
    cols = jax.lax.broadcasted_iota(jnp.int32, (n, n), 1)
    offdiag = rows != cols
    pref = jnp.where(
        offdiag & (u_raw < _HP2), jnp.sqrt(2.0 / _R_MAX) * env, 0.0)
    inv_r_m = jnp.where(offdiag, inv_r, 0.0)

    two_c1 = c1 + c1
    rbs = []
    t_prev = jnp.zeros_like(t1)
    t_cur = t1
    for _ in range(_BESSEL):
        rbs.append((pref * t_cur).astype(jnp.bfloat16))
        t_prev, t_cur = t_cur, two_c1 * t_cur - t_prev

    h_s = we_ref[:] + jnp.zeros((n, _F_S), f32)
    hv = jnp.zeros((n, 3 * _F_V), f32)
    inv_n = 1.0 / float(n)
    xc = [x[:, c:c + 1] for c in range(3)]
    xrep = jnp.concatenate(
        [jnp.broadcast_to(xc[c], (n, _F_V)) for c in range(3)], axis=1)
    zero88 = jnp.zeros((_F_V, _F_V), f32)

    for l in range(_L):
        w_rs = wrs_ref[l]
        w_rv = wrv_ref[l]
        S = jnp.concatenate([w_rs, w_rv, w_rv, w_rv], axis=1)
        M = jnp.concatenate([h_s, hv], axis=1)
        agg = _dot(rbs[0], (M * S[0:1, :]).astype(jnp.bfloat16))
        for b in range(1, _BESSEL):
            agg = agg + _dot(rbs[b], (M * S[b:b + 1, :]).astype(jnp.bfloat16))
        g = _dot(h_s, wvs_ref[l])
        P = jnp.concatenate([g * xc[0], g * xc[1], g * xc[2], g], axis=1)
        Q = _dot(inv_r_m, P)
        qg = jnp.concatenate([Q[:, 3 * _F_V:]] * 3, axis=1)
        av = (agg[:, _F_S:] + xrep * qg - Q[:, :3 * _F_V]) * inv_n
        h_s = h_s + _dot(agg[:, :_F_S] * inv_n, ws_ref[l])
        w_v = wv_ref[l]
        w_v_bd = jnp.concatenate([
            jnp.concatenate([w_v, zero88, zero88], axis=1),
            jnp.concatenate([zero88, w_v, zero88], axis=1),
            jnp.concatenate([zero88, zero88, w_v], axis=1)], axis=0)
        hv = hv + _dot(av, w_v_bd)

    sout_ref[:] = _dot(h_s, wrds_ref[:])
    wrdv = wrdv_ref[:]
    for c in range(3):
        com_c = jnp.sum(xT[c:c + 1, :], axis=1, keepdims=True) * inv_n
        vout_ref[c] = _dot(hv[:, _F_V * c:_F_V * (c + 1)], wrdv) + com_c


_call = pl.pallas_call(
    _mace_body,
    out_shape=[
        jax.ShapeDtypeStruct((3, _N, _F_V), jnp.float32),
        jax.ShapeDtypeStruct((_N, _F_S), jnp.float32),
    ],
    compiler_params=pltpu.CompilerParams(vmem_limit_bytes=100 * 1024 * 1024),
)


def kernel(x, W_embed, W_rad_s, W_rad_v, W_vs, W_s, W_v, W_read_s, W_read_v):
    xT = x.T
    vout, sout = _call(
        x, xT, W_embed, W_rad_s, W_rad_v, W_vs, W_s, W_v, W_read_s, W_read_v)
    return (jnp.transpose(vout, (1, 2, 0)), sout)
